# Initial kernel scaffold; baseline (speedup 1.0000x reference)
import jax
import jax.numpy as jnp
from jax.experimental import pallas as pl


def kernel(x, edge_attr, W0, b0, We1, be1, We2, be2, root, cb, Wg_ih, Wg_hh, bg_ih, bg_hh, Ws_ih, Ws_hh, bs_ih, bs_hh, Wm_ih, Wm_hh, bm_ih, bm_hh, W1, b1, W3, b3, edge_index, batch, nonring):
    raise NotImplementedError("write your pallas kernel here")


# speedup vs baseline: 3.3091x; 3.3091x over previous
import functools

import jax
import jax.numpy as jnp
from jax import lax
from jax.experimental import pallas as pl
from jax.experimental.pallas import tpu as pltpu
from jax.experimental.pallas import tpu_sc as plsc

N = 10000
E = 160000
D = 16

NC = 2
NS = 16
NW = NC * NS
CH = 128
NCH = 40
CPW = NCH * CH
E_PAD = NW * CPW
N_PAD = 10240
NPT = N_PAD // NS

BE = 2048
GE = E_PAD // BE

_F32 = jnp.float32


def _sc_mesh():
    return plsc.VectorSubcoreMesh(
        core_axis_name="c", subcore_axis_name="s", num_cores=NC, num_subcores=NS
    )


@functools.cache
def _build_sc_gather():
    @functools.partial(
        pl.kernel,
        out_type=jax.ShapeDtypeStruct((E_PAD, D), _F32),
        mesh=_sc_mesh(),
        scratch_types=[
            pltpu.VMEM((NCH, CH), jnp.int32),
            pltpu.VMEM((CPW, D), _F32),
            pltpu.SemaphoreType.DMA,
        ],
        compiler_params=pltpu.CompilerParams(use_tc_tiling_on_sc=False),
    )
    def sc_gather(table_h, idx_h, out_h, idx_v, rows_v, sem):
        w = lax.axis_index("s") * NC + lax.axis_index("c")
        pltpu.sync_copy(idx_h.at[w], idx_v)

        def body(j, carry):
            pltpu.async_copy(
                table_h.at[idx_v.at[j]], rows_v.at[pl.ds(j * CH, CH)], sem
            ).wait()
            return carry

        lax.fori_loop(0, NCH, body, 0)
        pltpu.sync_copy(rows_v, out_h.at[pl.ds(w * CPW, CPW)])

    return sc_gather


def _sc_gather(table, idx):
    return _build_sc_gather()(table, idx)


@functools.cache
def _build_sc_scatter():
    @functools.partial(
        pl.kernel,
        out_type=jax.ShapeDtypeStruct((NC, N_PAD, D), _F32),
        mesh=_sc_mesh(),
        scratch_types=[
            pltpu.VMEM((NCH, CH), jnp.int32),
            pltpu.VMEM((CPW, D), _F32),
            pltpu.VMEM_SHARED((N_PAD, D), _F32),
            pltpu.SemaphoreType.DMA,
        ],
        compiler_params=pltpu.CompilerParams(use_tc_tiling_on_sc=False),
    )
    def sc_scatter(msg_h, idx_h, zeros_h, out_h, idx_v, rows_v, agg_sh, sem):
        c = lax.axis_index("c")
        s = lax.axis_index("s")
        w = s * NC + c
        pltpu.sync_copy(zeros_h, agg_sh.at[pl.ds(s * NPT, NPT)])
        plsc.subcore_barrier()
        pltpu.sync_copy(idx_h.at[w], idx_v)
        pltpu.sync_copy(msg_h.at[pl.ds(w * CPW, CPW)], rows_v)

        def body(j, carry):
            pltpu.sync_copy(
                rows_v.at[pl.ds(j * CH, CH)], agg_sh.at[idx_v.at[j]], add=True
            )
            return carry

        lax.fori_loop(0, NCH, body, 0)
        plsc.subcore_barrier()
        pltpu.sync_copy(
            agg_sh.at[pl.ds(s * NPT, NPT)], out_h.at[c].at[pl.ds(s * NPT, NPT)]
        )

    return sc_scatter


def _sc_scatter(msg, idx, zeros_blk):
    return _build_sc_scatter()(msg, idx, zeros_blk)


def _lin0_body(x_ref, w_ref, b_ref, out_ref):
    out_ref[...] = jnp.maximum(
        jnp.dot(x_ref[...], w_ref[...], preferred_element_type=_F32) + b_ref[...],
        0.0,
    )


def _edge_net_body(ea_ref, w1_ref, b1_ref, w2_ref, b2_ref, out_ref):
    he = jnp.maximum(
        jnp.dot(ea_ref[...], w1_ref[...], preferred_element_type=_F32)
        + b1_ref[...],
        0.0,
    )
    out_ref[...] = (
        jnp.dot(he, w2_ref[...], preferred_element_type=_F32) + b2_ref[...]
    )


def _msg_body(w_ref, u_ref, out_ref):
    i1 = lax.broadcasted_iota(jnp.int32, (D, D * D), 0)
    j1 = lax.broadcasted_iota(jnp.int32, (D, D * D), 1)
    k1 = (i1 == j1 // D).astype(_F32)
    j2 = lax.broadcasted_iota(jnp.int32, (D * D, D), 0)
    o2 = lax.broadcasted_iota(jnp.int32, (D * D, D), 1)
    k2 = (j2 % D == o2).astype(_F32)
    uexp = jnp.dot(u_ref[...], k1, preferred_element_type=_F32)
    out_ref[...] = jnp.dot(
        w_ref[...] * uexp, k2, preferred_element_type=_F32
    )


def _gru_body(p0_ref, p1_ref, d0_ref, d1_ref, h_ref, root_ref, cb_ref,
              wih_ref, whh_ref, bih_ref, bhh_ref, out_ref):
    deg = d0_ref[...][:, :1] + d1_ref[...][:, :1]
    denom = jnp.maximum(deg, 1.0)
    agg = (p0_ref[...] + p1_ref[...]) / denom
    h = h_ref[...]
    m = jnp.maximum(
        agg + jnp.dot(h, root_ref[...], preferred_element_type=_F32)
        + cb_ref[...],
        0.0,
    )
    gi = jnp.dot(m, wih_ref[...], preferred_element_type=_F32) + bih_ref[...]
    gh = jnp.dot(h, whh_ref[...], preferred_element_type=_F32) + bhh_ref[...]
    r = jax.nn.sigmoid(gi[:, :D] + gh[:, :D])
    z = jax.nn.sigmoid(gi[:, D:2 * D] + gh[:, D:2 * D])
    n = jnp.tanh(gi[:, 2 * D:] + r * gh[:, 2 * D:])
    out_ref[...] = (1.0 - z) * n + z * h


def _s2s_body(out_ref, batch_ref, wsih_ref, wshh_ref, bsih_ref, bshh_ref,
              wmih_ref, bmih_ref, bmhh_ref, w1_ref, b1_ref, w3_ref, b3_ref,
              v_ref, hm_ref, cm_ref):
    outv = out_ref[...]
    oh = (batch_ref[...] == lax.broadcasted_iota(jnp.int32, (1, 64), 1))
    oh = oh.astype(_F32)
    qs = jnp.zeros((64, 2 * D), _F32)
    hs = jnp.zeros((64, D), _F32)
    cs = jnp.zeros((64, D), _F32)
    for _ in range(6):
        gates = (
            jnp.dot(qs, wsih_ref[...], preferred_element_type=_F32)
            + bsih_ref[...]
            + jnp.dot(hs, wshh_ref[...], preferred_element_type=_F32)
            + bshh_ref[...]
        )
        gi = gates[:, :D]
        gf = gates[:, D:2 * D]
        gg = gates[:, 2 * D:3 * D]
        go = gates[:, 3 * D:]
        cs = jax.nn.sigmoid(gf) * cs + jax.nn.sigmoid(gi) * jnp.tanh(gg)
        hs = jax.nn.sigmoid(go) * jnp.tanh(cs)
        qb = jnp.dot(oh, hs, preferred_element_type=_F32)
        e = jnp.sum(outv * qb, axis=1, keepdims=True)
        m = jnp.where(oh > 0.0, e, -jnp.inf)
        emax = jnp.max(m, axis=0, keepdims=True)
        emax = jnp.where(emax > -3e38, emax, 0.0)
        ee = jnp.exp(e - jnp.dot(oh, emax.T,
                                 preferred_element_type=_F32))
        den = lax.dot_general(oh, ee, (((0,), (0,)), ((), ())),
                              preferred_element_type=_F32)
        denb = jnp.dot(oh, den, preferred_element_type=_F32)
        a = ee / denb
        rvec = lax.dot_general(oh, a * outv, (((0,), (0,)), ((), ())),
                               preferred_element_type=_F32)
        qs = jnp.concatenate([hs, rvec], axis=1)
    gates = (
        jnp.dot(qs, wmih_ref[...], preferred_element_type=_F32)
        + bmih_ref[...] + bmhh_ref[...]
    )
    gi = gates[:, :2 * D]
    gf = gates[:, 2 * D:4 * D]
    gg = gates[:, 4 * D:6 * D]
    go = gates[:, 6 * D:]
    cm = jax.nn.sigmoid(gi) * jnp.tanh(gg)
    hm = jax.nn.sigmoid(go) * jnp.tanh(cm)
    out2 = jnp.maximum(
        jnp.dot(hm, w1_ref[...], preferred_element_type=_F32) + b1_ref[...],
        0.0,
    )
    v_ref[...] = (
        jnp.dot(out2, w3_ref[...], preferred_element_type=_F32) + b3_ref[...]
    )
    hm_ref[...] = hm
    cm_ref[...] = cm


def kernel(x, edge_attr, W0, b0, We1, be1, We2, be2, root, cb,
           Wg_ih, Wg_hh, bg_ih, bg_hh, Ws_ih, Ws_hh, bs_ih, bs_hh,
           Wm_ih, Wm_hh, bm_ih, bm_hh, W1, b1, W3, b3,
           edge_index, batch, nonring):
    pad = E_PAD - E
    src_p = jnp.concatenate(
        [edge_index[0], jnp.zeros((pad,), jnp.int32)]
    ).reshape(NW, NCH, CH)
    dst_p = jnp.concatenate(
        [edge_index[1], jnp.full((pad,), N, jnp.int32)]
    ).reshape(NW, NCH, CH)
    ea_p = jnp.pad(edge_attr, ((0, pad), (0, 1)))
    We1_p = jnp.pad(We1, ((0, 1), (0, 0)))
    zeros_blk = jnp.zeros((NPT, D), _F32)

    lin0 = pl.pallas_call(
        _lin0_body, out_shape=jax.ShapeDtypeStruct((N, D), _F32)
    )
    h = lin0(x, W0, b0.reshape(1, D))

    edge_net = pl.pallas_call(
        _edge_net_body,
        grid=(GE,),
        in_specs=[
            pl.BlockSpec((BE, 8), lambda i: (i, 0)),
            pl.BlockSpec((8, 128), lambda i: (0, 0)),
            pl.BlockSpec((1, 128), lambda i: (0, 0)),
            pl.BlockSpec((128, D * D), lambda i: (0, 0)),
            pl.BlockSpec((1, D * D), lambda i: (0, 0)),
        ],
        out_specs=pl.BlockSpec((BE, D * D), lambda i: (i, 0)),
        out_shape=jax.ShapeDtypeStruct((E_PAD, D * D), _F32),
    )
    w_e = edge_net(ea_p, We1_p, be1.reshape(1, 128), We2, be2.reshape(1, D * D))

    msg_call = pl.pallas_call(
        _msg_body,
        grid=(GE,),
        in_specs=[
            pl.BlockSpec((BE, D * D), lambda i: (i, 0)),
            pl.BlockSpec((BE, D), lambda i: (i, 0)),
        ],
        out_specs=pl.BlockSpec((BE, D), lambda i: (i, 0)),
        out_shape=jax.ShapeDtypeStruct((E_PAD, D), _F32),
    )

    gru_call = pl.pallas_call(
        _gru_body, out_shape=jax.ShapeDtypeStruct((N, D), _F32)
    )

    ones_msg = jnp.ones((E_PAD, D), _F32)
    degp = _sc_scatter(ones_msg, dst_p, zeros_blk)
    d0 = degp[0, :N]
    d1 = degp[1, :N]

    for _ in range(6):
        u = _sc_gather(h, src_p)
        msg = msg_call(w_e, u)
        aggp = _sc_scatter(msg, dst_p, zeros_blk)
        h = gru_call(
            aggp[0, :N], aggp[1, :N], d0, d1, h,
            root, cb.reshape(1, D),
            Wg_ih, Wg_hh, bg_ih.reshape(1, 3 * D), bg_hh.reshape(1, 3 * D),
        )

    s2s = pl.pallas_call(
        _s2s_body,
        out_shape=(
            jax.ShapeDtypeStruct((64, 1), _F32),
            jax.ShapeDtypeStruct((64, 2 * D), _F32),
            jax.ShapeDtypeStruct((64, 2 * D), _F32),
        ),
    )
    v, hm, cm = s2s(
        h, batch.reshape(N, 1),
        Ws_ih, Ws_hh, bs_ih.reshape(1, 4 * D), bs_hh.reshape(1, 4 * D),
        Wm_ih, bm_ih.reshape(1, 8 * D), bm_hh.reshape(1, 8 * D),
        W1, b1.reshape(1, D), W3, b3.reshape(1, 1),
    )
    return (v, hm[None], cm[None])


# speedup vs baseline: 3.5516x; 1.0733x over previous
import functools

import jax
import jax.numpy as jnp
from jax import lax
from jax.experimental import pallas as pl
from jax.experimental.pallas import tpu as pltpu
from jax.experimental.pallas import tpu_sc as plsc

N = 10000
E = 160000
D = 16

NC = 2
NS = 16
NW = NC * NS
CH = 128
NCH = 40
CPW = NCH * CH
E_PAD = NW * CPW
N_PAD = 10240
NPT = N_PAD // NS

BE = 2048
GE = E_PAD // BE
EP8 = E_PAD // 8

_F32 = jnp.float32
_BF16 = jnp.bfloat16


def _sc_mesh():
    return plsc.VectorSubcoreMesh(
        core_axis_name="c", subcore_axis_name="s", num_cores=NC, num_subcores=NS
    )


@functools.cache
def _build_sc_gather():
    @functools.partial(
        pl.kernel,
        out_type=jax.ShapeDtypeStruct((E_PAD, D), _F32),
        mesh=_sc_mesh(),
        scratch_types=[
            pltpu.VMEM((NCH, CH), jnp.int32),
            pltpu.VMEM((CPW, D), _F32),
            pltpu.SemaphoreType.DMA,
        ],
        compiler_params=pltpu.CompilerParams(use_tc_tiling_on_sc=False),
    )
    def sc_gather(table_h, idx_h, out_h, idx_v, rows_v, sem):
        w = lax.axis_index("s") * NC + lax.axis_index("c")
        pltpu.sync_copy(idx_h.at[w], idx_v)

        def fire(j, carry):
            pltpu.async_copy(
                table_h.at[idx_v.at[j]], rows_v.at[pl.ds(j * CH, CH)], sem
            )
            return carry

        lax.fori_loop(0, NCH, fire, 0)
        pltpu.make_async_copy(table_h.at[pl.ds(0, CPW)], rows_v, sem).wait()
        pltpu.sync_copy(rows_v, out_h.at[pl.ds(w * CPW, CPW)])

    return sc_gather


def _sc_gather(table, idx):
    return _build_sc_gather()(table, idx)


@functools.cache
def _build_sc_scatter():
    @functools.partial(
        pl.kernel,
        out_type=jax.ShapeDtypeStruct((NC, N_PAD, D), _F32),
        mesh=_sc_mesh(),
        scratch_types=[
            pltpu.VMEM((NCH, CH), jnp.int32),
            pltpu.VMEM((CPW, D), _F32),
            pltpu.VMEM_SHARED((N_PAD, D), _F32),
            pltpu.SemaphoreType.DMA,
            pltpu.SemaphoreType.DMA,
        ],
        compiler_params=pltpu.CompilerParams(use_tc_tiling_on_sc=False),
    )
    def sc_scatter(msg_h, idx_h, zeros_h, out_h, idx_v, rows_v, agg_sh, sem,
                   sem2):
        c = lax.axis_index("c")
        s = lax.axis_index("s")
        w = s * NC + c
        pltpu.sync_copy(zeros_h, agg_sh.at[pl.ds(s * NPT, NPT)])
        pltpu.sync_copy(idx_h.at[w], idx_v)
        pltpu.sync_copy(msg_h.at[pl.ds(w * CPW, CPW)], rows_v)
        plsc.subcore_barrier()

        def fire(j, carry):
            pltpu.async_copy(
                rows_v.at[pl.ds(j * CH, CH)], agg_sh.at[idx_v.at[j]], sem2,
                add=True,
            )
            return carry

        lax.fori_loop(0, NCH, fire, 0)
        pltpu.make_async_copy(msg_h.at[pl.ds(0, CPW)], rows_v, sem2).wait()
        plsc.subcore_barrier()
        pltpu.sync_copy(
            agg_sh.at[pl.ds(s * NPT, NPT)], out_h.at[c].at[pl.ds(s * NPT, NPT)]
        )

    return sc_scatter


def _sc_scatter(msg, idx, zeros_blk):
    return _build_sc_scatter()(msg, idx, zeros_blk)


def _lin0_body(x_ref, w_ref, b_ref, out_ref):
    out_ref[...] = jnp.maximum(
        jnp.dot(x_ref[...], w_ref[...], preferred_element_type=_F32) + b_ref[...],
        0.0,
    )


def _edge_net_body(ea_ref, w1_ref, b1_ref, w2_ref, b2_ref, out_ref):
    he = jnp.maximum(
        jnp.dot(ea_ref[...], w1_ref[...], preferred_element_type=_F32)
        + b1_ref[...],
        0.0,
    )
    w = jnp.dot(he, w2_ref[...], preferred_element_type=_F32) + b2_ref[...]
    out_ref[...] = w.astype(_BF16)


def _msg_body(w_ref, u_ref, out_ref):
    i1 = lax.broadcasted_iota(jnp.int32, (D, D * D), 0)
    j1 = lax.broadcasted_iota(jnp.int32, (D, D * D), 1)
    k1 = (i1 == j1 // D).astype(_F32)
    j2 = lax.broadcasted_iota(jnp.int32, (D * D, D), 0)
    o2 = lax.broadcasted_iota(jnp.int32, (D * D, D), 1)
    k2 = (j2 % D == o2).astype(_F32)
    uexp = jnp.dot(u_ref[...], k1, preferred_element_type=_F32)
    w = w_ref[...].astype(_F32)
    out_ref[...] = jnp.dot(w * uexp, k2, preferred_element_type=_F32)


def _gru_body(p0_ref, p1_ref, d0_ref, d1_ref, h_ref, root_ref, cb_ref,
              wih_ref, whh_ref, bih_ref, bhh_ref, out_ref):
    deg = d0_ref[...][:, :1] + d1_ref[...][:, :1]
    denom = jnp.maximum(deg, 1.0)
    agg = (p0_ref[...] + p1_ref[...]) / denom
    h = h_ref[...]
    m = jnp.maximum(
        agg + jnp.dot(h, root_ref[...], preferred_element_type=_F32)
        + cb_ref[...],
        0.0,
    )
    gi = jnp.dot(m, wih_ref[...], preferred_element_type=_F32) + bih_ref[...]
    gh = jnp.dot(h, whh_ref[...], preferred_element_type=_F32) + bhh_ref[...]
    r = jax.nn.sigmoid(gi[:, :D] + gh[:, :D])
    z = jax.nn.sigmoid(gi[:, D:2 * D] + gh[:, D:2 * D])
    n = jnp.tanh(gi[:, 2 * D:] + r * gh[:, 2 * D:])
    out_ref[...] = (1.0 - z) * n + z * h


def _s2s_body(out_ref, batch_ref, wsih_ref, wshh_ref, bsih_ref, bshh_ref,
              wmih_ref, bmih_ref, bmhh_ref, w1_ref, b1_ref, w3_ref, b3_ref,
              v_ref, hm_ref, cm_ref):
    outv = out_ref[...]
    oh = (batch_ref[...] == lax.broadcasted_iota(jnp.int32, (1, 64), 1))
    oh = oh.astype(_F32)
    qs = jnp.zeros((64, 2 * D), _F32)
    hs = jnp.zeros((64, D), _F32)
    cs = jnp.zeros((64, D), _F32)
    for _ in range(6):
        gates = (
            jnp.dot(qs, wsih_ref[...], preferred_element_type=_F32)
            + bsih_ref[...]
            + jnp.dot(hs, wshh_ref[...], preferred_element_type=_F32)
            + bshh_ref[...]
        )
        gi = gates[:, :D]
        gf = gates[:, D:2 * D]
        gg = gates[:, 2 * D:3 * D]
        go = gates[:, 3 * D:]
        cs = jax.nn.sigmoid(gf) * cs + jax.nn.sigmoid(gi) * jnp.tanh(gg)
        hs = jax.nn.sigmoid(go) * jnp.tanh(cs)
        qb = jnp.dot(oh, hs, preferred_element_type=_F32)
        e = jnp.sum(outv * qb, axis=1, keepdims=True)
        m = jnp.where(oh > 0.0, e, -jnp.inf)
        emax = jnp.max(m, axis=0, keepdims=True)
        emax = jnp.where(emax > -3e38, emax, 0.0)
        ee = jnp.exp(e - jnp.dot(oh, emax.T,
                                 preferred_element_type=_F32))
        den = lax.dot_general(oh, ee, (((0,), (0,)), ((), ())),
                              preferred_element_type=_F32)
        denb = jnp.dot(oh, den, preferred_element_type=_F32)
        a = ee / denb
        rvec = lax.dot_general(oh, a * outv, (((0,), (0,)), ((), ())),
                               preferred_element_type=_F32)
        qs = jnp.concatenate([hs, rvec], axis=1)
    gates = (
        jnp.dot(qs, wmih_ref[...], preferred_element_type=_F32)
        + bmih_ref[...] + bmhh_ref[...]
    )
    gi = gates[:, :2 * D]
    gf = gates[:, 2 * D:4 * D]
    gg = gates[:, 4 * D:6 * D]
    go = gates[:, 6 * D:]
    cm = jax.nn.sigmoid(gi) * jnp.tanh(gg)
    hm = jax.nn.sigmoid(go) * jnp.tanh(cm)
    out2 = jnp.maximum(
        jnp.dot(hm, w1_ref[...], preferred_element_type=_F32) + b1_ref[...],
        0.0,
    )
    v_ref[...] = (
        jnp.dot(out2, w3_ref[...], preferred_element_type=_F32) + b3_ref[...]
    )
    hm_ref[...] = hm
    cm_ref[...] = cm


def kernel(x, edge_attr, W0, b0, We1, be1, We2, be2, root, cb,
           Wg_ih, Wg_hh, bg_ih, bg_hh, Ws_ih, Ws_hh, bs_ih, bs_hh,
           Wm_ih, Wm_hh, bm_ih, bm_hh, W1, b1, W3, b3,
           edge_index, batch, nonring):
    pad = E_PAD - E
    src_p = jnp.concatenate(
        [edge_index[0], jnp.zeros((pad,), jnp.int32)]
    ).reshape(NW, NCH, CH)
    dst_p = jnp.concatenate(
        [edge_index[1], jnp.full((pad,), N, jnp.int32)]
    ).reshape(NW, NCH, CH)
    ea_p = jnp.pad(edge_attr, ((0, pad), (0, 1)))
    We1_p = jnp.pad(We1, ((0, 1), (0, 0)))
    zeros_blk = jnp.zeros((NPT, D), _F32)

    lin0 = pl.pallas_call(
        _lin0_body, out_shape=jax.ShapeDtypeStruct((N, D), _F32)
    )
    h = lin0(x, W0, b0.reshape(1, D))

    edge_net = pl.pallas_call(
        _edge_net_body,
        grid=(GE,),
        in_specs=[
            pl.BlockSpec((BE, 8), lambda i: (i, 0)),
            pl.BlockSpec((8, 128), lambda i: (0, 0)),
            pl.BlockSpec((1, 128), lambda i: (0, 0)),
            pl.BlockSpec((128, D * D), lambda i: (0, 0)),
            pl.BlockSpec((1, D * D), lambda i: (0, 0)),
        ],
        out_specs=pl.BlockSpec((BE, D * D), lambda i: (i, 0)),
        out_shape=jax.ShapeDtypeStruct((E_PAD, D * D), _BF16),
    )
    w_e = edge_net(ea_p, We1_p, be1.reshape(1, 128), We2, be2.reshape(1, D * D))

    msg_call = pl.pallas_call(
        _msg_body,
        grid=(GE,),
        in_specs=[
            pl.BlockSpec((BE, D * D), lambda i: (i, 0)),
            pl.BlockSpec((BE, D), lambda i: (i, 0)),
        ],
        out_specs=pl.BlockSpec((BE, D), lambda i: (i, 0)),
        out_shape=jax.ShapeDtypeStruct((E_PAD, D), _F32),
    )

    gru_call = pl.pallas_call(
        _gru_body, out_shape=jax.ShapeDtypeStruct((N, D), _F32)
    )

    ones_msg = jnp.ones((E_PAD, D), _F32)
    degp = _sc_scatter(ones_msg, dst_p, zeros_blk)
    d0 = degp[0, :N]
    d1 = degp[1, :N]

    for _ in range(6):
        u = _sc_gather(h, src_p)
        msg = msg_call(w_e, u)
        aggp = _sc_scatter(msg, dst_p, zeros_blk)
        h = gru_call(
            aggp[0, :N], aggp[1, :N], d0, d1, h,
            root, cb.reshape(1, D),
            Wg_ih, Wg_hh, bg_ih.reshape(1, 3 * D), bg_hh.reshape(1, 3 * D),
        )

    s2s = pl.pallas_call(
        _s2s_body,
        out_shape=(
            jax.ShapeDtypeStruct((64, 1), _F32),
            jax.ShapeDtypeStruct((64, 2 * D), _F32),
            jax.ShapeDtypeStruct((64, 2 * D), _F32),
        ),
    )
    v, hm, cm = s2s(
        h, batch.reshape(N, 1),
        Ws_ih, Ws_hh, bs_ih.reshape(1, 4 * D), bs_hh.reshape(1, 4 * D),
        Wm_ih, bm_ih.reshape(1, 8 * D), bm_hh.reshape(1, 8 * D),
        W1, b1.reshape(1, D), W3, b3.reshape(1, 1),
    )
    return (v, hm[None], cm[None])


# speedup vs baseline: 5.3162x; 1.4968x over previous
import functools

import jax
import jax.numpy as jnp
from jax import lax
from jax.experimental import pallas as pl
from jax.experimental.pallas import tpu as pltpu
from jax.experimental.pallas import tpu_sc as plsc

N = 10000
E = 160000
D = 16

NC = 2
NS = 16
NW = NC * NS
CH = 128
NCH = 40
CPW = NCH * CH
E_PAD = NW * CPW
N_PAD = 10240
NPT = N_PAD // NS

BE = 2048
GE = E_PAD // BE
EP8 = E_PAD // 8

_F32 = jnp.float32
_BF16 = jnp.bfloat16


def _sc_mesh():
    return plsc.VectorSubcoreMesh(
        core_axis_name="c", subcore_axis_name="s", num_cores=NC, num_subcores=NS
    )


@functools.cache
def _build_sc_gather():
    @functools.partial(
        pl.kernel,
        out_type=jax.ShapeDtypeStruct((E_PAD, D), _F32),
        mesh=_sc_mesh(),
        scratch_types=[
            pltpu.VMEM((NCH, CH), jnp.int32),
            pltpu.VMEM((CPW, D), _F32),
            pltpu.SemaphoreType.DMA,
        ],
        compiler_params=pltpu.CompilerParams(use_tc_tiling_on_sc=False),
    )
    def sc_gather(table_h, idx_h, out_h, idx_v, rows_v, sem):
        w = lax.axis_index("s") * NC + lax.axis_index("c")
        pltpu.sync_copy(idx_h.at[w], idx_v)

        def fire(j, carry):
            pltpu.async_copy(
                table_h.at[idx_v.at[j]], rows_v.at[pl.ds(j * CH, CH)], sem
            )
            return carry

        lax.fori_loop(0, NCH, fire, 0)
        pltpu.make_async_copy(table_h.at[pl.ds(0, CPW)], rows_v, sem).wait()
        pltpu.sync_copy(rows_v, out_h.at[pl.ds(w * CPW, CPW)])

    return sc_gather


def _sc_gather(table, idx):
    return _build_sc_gather()(table, idx)


@functools.cache
def _build_sc_scatter():
    @functools.partial(
        pl.kernel,
        out_type=jax.ShapeDtypeStruct((NC, N_PAD, D), _F32),
        mesh=_sc_mesh(),
        scratch_types=[
            pltpu.VMEM((NCH, CH), jnp.int32),
            pltpu.VMEM((CPW, D), _F32),
            pltpu.VMEM_SHARED((N_PAD, D), _F32),
            pltpu.SemaphoreType.DMA,
            pltpu.SemaphoreType.DMA,
        ],
        compiler_params=pltpu.CompilerParams(use_tc_tiling_on_sc=False),
    )
    def sc_scatter(msg_h, idx_h, zeros_h, out_h, idx_v, rows_v, agg_sh, sem,
                   sem2):
        c = lax.axis_index("c")
        s = lax.axis_index("s")
        w = s * NC + c
        pltpu.sync_copy(zeros_h, agg_sh.at[pl.ds(s * NPT, NPT)])
        pltpu.sync_copy(idx_h.at[w], idx_v)
        pltpu.sync_copy(msg_h.at[pl.ds(w * CPW, CPW)], rows_v)
        plsc.subcore_barrier()

        def fire(j, carry):
            pltpu.async_copy(
                rows_v.at[pl.ds(j * CH, CH)], agg_sh.at[idx_v.at[j]], sem2,
                add=True,
            )
            return carry

        lax.fori_loop(0, NCH, fire, 0)
        pltpu.make_async_copy(msg_h.at[pl.ds(0, CPW)], rows_v, sem2).wait()
        plsc.subcore_barrier()
        pltpu.sync_copy(
            agg_sh.at[pl.ds(s * NPT, NPT)], out_h.at[c].at[pl.ds(s * NPT, NPT)]
        )

    return sc_scatter


def _sc_scatter(msg, idx, zeros_blk):
    return _build_sc_scatter()(msg, idx, zeros_blk)


@functools.cache
def _build_sc_fused():
    @functools.partial(
        pl.kernel,
        out_type=jax.ShapeDtypeStruct((NC, N_PAD, D), _F32),
        mesh=_sc_mesh(),
        scratch_types=[
            pltpu.VMEM((NCH, CH), jnp.int32),
            pltpu.VMEM((NCH, CH), jnp.int32),
            pltpu.VMEM((CH, D), _F32),
            pltpu.VMEM((CH, D), _F32),
            pltpu.VMEM((CH, D * D), _F32),
            pltpu.VMEM((CH, D * D), _F32),
            pltpu.VMEM((CH, D), _F32),
            pltpu.VMEM((CH, D), _F32),
            pltpu.VMEM_SHARED((N_PAD, D), _F32),
            pltpu.SemaphoreType.DMA,
            pltpu.SemaphoreType.DMA,
            pltpu.SemaphoreType.DMA,
            pltpu.SemaphoreType.DMA,
            pltpu.SemaphoreType.DMA,
        ],
        compiler_params=pltpu.CompilerParams(use_tc_tiling_on_sc=False),
    )
    def sc_fused(table_h, w_h, didx_h, sidx_h, zeros_h, out_h,
                 didx_v, sidx_v, u0, u1, w0, w1, m0, m1, agg_sh,
                 su0, su1, sw0, sw1, ssc):
        c = lax.axis_index("c")
        s = lax.axis_index("s")
        wid = s * NC + c
        pltpu.sync_copy(zeros_h, agg_sh.at[pl.ds(s * NPT, NPT)])
        pltpu.sync_copy(didx_h.at[wid], didx_v)
        pltpu.sync_copy(sidx_h.at[wid], sidx_v)
        plsc.subcore_barrier()
        base = wid * CPW
        ubufs = (u0, u1)
        wbufs = (w0, w1)
        mbufs = (m0, m1)
        usems = (su0, su1)
        wsems = (sw0, sw1)

        def issue(j, b):
            pltpu.async_copy(table_h.at[sidx_v.at[j]], ubufs[b], usems[b])
            pltpu.async_copy(
                w_h.at[pl.ds(base + j * CH, CH)], wbufs[b], wsems[b]
            )

        issue(0, 0)

        def pair(p, carry):
            for b in (0, 1):
                j = 2 * p + b

                @pl.when(j + 1 < NCH)
                def _():
                    issue(j + 1, 1 - b)

                pltpu.make_async_copy(
                    table_h.at[pl.ds(0, CH)], ubufs[b], usems[b]
                ).wait()
                pltpu.make_async_copy(
                    w_h.at[pl.ds(0, CH)], wbufs[b], wsems[b]
                ).wait()

                @pl.when(j >= 2)
                def _():
                    pltpu.make_async_copy(
                        table_h.at[pl.ds(0, CH)], mbufs[b], ssc
                    ).wait()

                def edge_body(e, carry2):
                    uvec = ubufs[b][e, :]
                    acc = None
                    for i in range(D):
                        ui = uvec.at[
                            jnp.full((D,), i, jnp.int32)
                        ].get(mode="promise_in_bounds")
                        wrow = wbufs[b][e, pl.ds(i * D, D)]
                        t = ui * wrow
                        acc = t if acc is None else acc + t
                    mbufs[b][e, :] = acc
                    return carry2

                lax.fori_loop(0, CH, edge_body, 0)
                pltpu.async_copy(
                    mbufs[b], agg_sh.at[didx_v.at[j]], ssc, add=True
                )
            return carry

        lax.fori_loop(0, NCH // 2, pair, 0)
        pltpu.make_async_copy(table_h.at[pl.ds(0, CH)], m0, ssc).wait()
        pltpu.make_async_copy(table_h.at[pl.ds(0, CH)], m1, ssc).wait()
        plsc.subcore_barrier()
        pltpu.sync_copy(
            agg_sh.at[pl.ds(s * NPT, NPT)], out_h.at[c].at[pl.ds(s * NPT, NPT)]
        )

    return sc_fused


def _sc_fused(table, w_e, didx, sidx, zeros_blk):
    return _build_sc_fused()(table, w_e, didx, sidx, zeros_blk)


def _lin0_body(x_ref, w_ref, b_ref, out_ref):
    out_ref[...] = jnp.maximum(
        jnp.dot(x_ref[...], w_ref[...], preferred_element_type=_F32) + b_ref[...],
        0.0,
    )


def _edge_net_body(ea_ref, w1_ref, b1_ref, w2_ref, b2_ref, out_ref):
    he = jnp.maximum(
        jnp.dot(ea_ref[...], w1_ref[...], preferred_element_type=_F32)
        + b1_ref[...],
        0.0,
    )
    out_ref[...] = (
        jnp.dot(he, w2_ref[...], preferred_element_type=_F32) + b2_ref[...]
    )


def _msg_body(w_ref, u_ref, out_ref):
    i1 = lax.broadcasted_iota(jnp.int32, (D, D * D), 0)
    j1 = lax.broadcasted_iota(jnp.int32, (D, D * D), 1)
    k1 = (i1 == j1 // D).astype(_F32)
    j2 = lax.broadcasted_iota(jnp.int32, (D * D, D), 0)
    o2 = lax.broadcasted_iota(jnp.int32, (D * D, D), 1)
    k2 = (j2 % D == o2).astype(_F32)
    uexp = jnp.dot(u_ref[...], k1, preferred_element_type=_F32)
    w = w_ref[...].astype(_F32)
    out_ref[...] = jnp.dot(w * uexp, k2, preferred_element_type=_F32)


def _gru_body(p0_ref, p1_ref, d0_ref, d1_ref, h_ref, root_ref, cb_ref,
              wih_ref, whh_ref, bih_ref, bhh_ref, out_ref):
    deg = d0_ref[...][:, :1] + d1_ref[...][:, :1]
    denom = jnp.maximum(deg, 1.0)
    agg = (p0_ref[...] + p1_ref[...]) / denom
    h = h_ref[...]
    m = jnp.maximum(
        agg + jnp.dot(h, root_ref[...], preferred_element_type=_F32)
        + cb_ref[...],
        0.0,
    )
    gi = jnp.dot(m, wih_ref[...], preferred_element_type=_F32) + bih_ref[...]
    gh = jnp.dot(h, whh_ref[...], preferred_element_type=_F32) + bhh_ref[...]
    r = jax.nn.sigmoid(gi[:, :D] + gh[:, :D])
    z = jax.nn.sigmoid(gi[:, D:2 * D] + gh[:, D:2 * D])
    n = jnp.tanh(gi[:, 2 * D:] + r * gh[:, 2 * D:])
    out_ref[...] = (1.0 - z) * n + z * h


def _s2s_body(out_ref, batch_ref, wsih_ref, wshh_ref, bsih_ref, bshh_ref,
              wmih_ref, bmih_ref, bmhh_ref, w1_ref, b1_ref, w3_ref, b3_ref,
              v_ref, hm_ref, cm_ref):
    outv = out_ref[...]
    oh = (batch_ref[...] == lax.broadcasted_iota(jnp.int32, (1, 64), 1))
    oh = oh.astype(_F32)
    qs = jnp.zeros((64, 2 * D), _F32)
    hs = jnp.zeros((64, D), _F32)
    cs = jnp.zeros((64, D), _F32)
    for _ in range(6):
        gates = (
            jnp.dot(qs, wsih_ref[...], preferred_element_type=_F32)
            + bsih_ref[...]
            + jnp.dot(hs, wshh_ref[...], preferred_element_type=_F32)
            + bshh_ref[...]
        )
        gi = gates[:, :D]
        gf = gates[:, D:2 * D]
        gg = gates[:, 2 * D:3 * D]
        go = gates[:, 3 * D:]
        cs = jax.nn.sigmoid(gf) * cs + jax.nn.sigmoid(gi) * jnp.tanh(gg)
        hs = jax.nn.sigmoid(go) * jnp.tanh(cs)
        qb = jnp.dot(oh, hs, preferred_element_type=_F32)
        e = jnp.sum(outv * qb, axis=1, keepdims=True)
        m = jnp.where(oh > 0.0, e, -jnp.inf)
        emax = jnp.max(m, axis=0, keepdims=True)
        emax = jnp.where(emax > -3e38, emax, 0.0)
        ee = jnp.exp(e - jnp.dot(oh, emax.T,
                                 preferred_element_type=_F32))
        den = lax.dot_general(oh, ee, (((0,), (0,)), ((), ())),
                              preferred_element_type=_F32)
        denb = jnp.dot(oh, den, preferred_element_type=_F32)
        a = ee / denb
        rvec = lax.dot_general(oh, a * outv, (((0,), (0,)), ((), ())),
                               preferred_element_type=_F32)
        qs = jnp.concatenate([hs, rvec], axis=1)
    gates = (
        jnp.dot(qs, wmih_ref[...], preferred_element_type=_F32)
        + bmih_ref[...] + bmhh_ref[...]
    )
    gi = gates[:, :2 * D]
    gf = gates[:, 2 * D:4 * D]
    gg = gates[:, 4 * D:6 * D]
    go = gates[:, 6 * D:]
    cm = jax.nn.sigmoid(gi) * jnp.tanh(gg)
    hm = jax.nn.sigmoid(go) * jnp.tanh(cm)
    out2 = jnp.maximum(
        jnp.dot(hm, w1_ref[...], preferred_element_type=_F32) + b1_ref[...],
        0.0,
    )
    v_ref[...] = (
        jnp.dot(out2, w3_ref[...], preferred_element_type=_F32) + b3_ref[...]
    )
    hm_ref[...] = hm
    cm_ref[...] = cm


def kernel(x, edge_attr, W0, b0, We1, be1, We2, be2, root, cb,
           Wg_ih, Wg_hh, bg_ih, bg_hh, Ws_ih, Ws_hh, bs_ih, bs_hh,
           Wm_ih, Wm_hh, bm_ih, bm_hh, W1, b1, W3, b3,
           edge_index, batch, nonring):
    pad = E_PAD - E
    src_p = jnp.concatenate(
        [edge_index[0], jnp.zeros((pad,), jnp.int32)]
    ).reshape(NW, NCH, CH)
    dst_p = jnp.concatenate(
        [edge_index[1], jnp.full((pad,), N, jnp.int32)]
    ).reshape(NW, NCH, CH)
    ea_p = jnp.pad(edge_attr, ((0, pad), (0, 1)))
    We1_p = jnp.pad(We1, ((0, 1), (0, 0)))
    zeros_blk = jnp.zeros((NPT, D), _F32)

    lin0 = pl.pallas_call(
        _lin0_body, out_shape=jax.ShapeDtypeStruct((N, D), _F32)
    )
    h = lin0(x, W0, b0.reshape(1, D))

    edge_net = pl.pallas_call(
        _edge_net_body,
        grid=(GE,),
        in_specs=[
            pl.BlockSpec((BE, 8), lambda i: (i, 0)),
            pl.BlockSpec((8, 128), lambda i: (0, 0)),
            pl.BlockSpec((1, 128), lambda i: (0, 0)),
            pl.BlockSpec((128, D * D), lambda i: (0, 0)),
            pl.BlockSpec((1, D * D), lambda i: (0, 0)),
        ],
        out_specs=pl.BlockSpec((BE, D * D), lambda i: (i, 0)),
        out_shape=jax.ShapeDtypeStruct((E_PAD, D * D), _F32),
    )
    w_e = edge_net(ea_p, We1_p, be1.reshape(1, 128), We2, be2.reshape(1, D * D))

    msg_call = pl.pallas_call(
        _msg_body,
        grid=(GE,),
        in_specs=[
            pl.BlockSpec((BE, D * D), lambda i: (i, 0)),
            pl.BlockSpec((BE, D), lambda i: (i, 0)),
        ],
        out_specs=pl.BlockSpec((BE, D), lambda i: (i, 0)),
        out_shape=jax.ShapeDtypeStruct((E_PAD, D), _F32),
    )

    gru_call = pl.pallas_call(
        _gru_body, out_shape=jax.ShapeDtypeStruct((N, D), _F32)
    )

    ones_msg = jnp.ones((E_PAD, D), _F32)
    degp = _sc_scatter(ones_msg, dst_p, zeros_blk)
    d0 = degp[0, :N]
    d1 = degp[1, :N]

    for _ in range(6):
        aggp = _sc_fused(h, w_e, dst_p, src_p, zeros_blk)
        h = gru_call(
            aggp[0, :N], aggp[1, :N], d0, d1, h,
            root, cb.reshape(1, D),
            Wg_ih, Wg_hh, bg_ih.reshape(1, 3 * D), bg_hh.reshape(1, 3 * D),
        )

    s2s = pl.pallas_call(
        _s2s_body,
        out_shape=(
            jax.ShapeDtypeStruct((64, 1), _F32),
            jax.ShapeDtypeStruct((64, 2 * D), _F32),
            jax.ShapeDtypeStruct((64, 2 * D), _F32),
        ),
    )
    v, hm, cm = s2s(
        h, batch.reshape(N, 1),
        Ws_ih, Ws_hh, bs_ih.reshape(1, 4 * D), bs_hh.reshape(1, 4 * D),
        Wm_ih, bm_ih.reshape(1, 8 * D), bm_hh.reshape(1, 8 * D),
        W1, b1.reshape(1, D), W3, b3.reshape(1, 1),
    )
    return (v, hm[None], cm[None])
